# baseline (speedup 1.0000x reference)
import functools

import jax
import jax.numpy as jnp
from jax import lax
from jax.experimental import pallas as pl
from jax.experimental.pallas import tpu as pltpu
from jax.experimental.pallas import tpu_sc as plsc

NC = 2
NS = 16
NW = NC * NS
CH = 128


@functools.lru_cache(maxsize=None)
def _build(n_rows, d, n_sup):
    assert n_rows % (NW * CH) == 0
    b_per_w = n_rows // NW
    n_chunks = b_per_w // CH
    mesh = plsc.VectorSubcoreMesh(core_axis_name="c", subcore_axis_name="s")

    @functools.partial(
        pl.kernel,
        mesh=mesh,
        compiler_params=pltpu.CompilerParams(needs_layout_passes=False),
        out_type=jax.ShapeDtypeStruct((n_rows, d), jnp.float32),
        scratch_types=[
            pltpu.VMEM((n_chunks, CH), jnp.int32),
            pltpu.VMEM((n_chunks, CH), jnp.int32),
            pltpu.VMEM((CH, 128), jnp.float32),
            pltpu.VMEM((CH, d), jnp.float32),
            pltpu.SemaphoreType.DMA,
            pltpu.SemaphoreType.DMA,
        ],
    )
    def body(sidx_hbm, ofs_hbm, table_hbm, out_hbm,
             sidx_v, ofs_v, sup_v, out_v, gsem, ssem):
        wid = lax.axis_index("s") * NC + lax.axis_index("c")
        cbase = wid * n_chunks
        rbase = wid * b_per_w
        pltpu.sync_copy(sidx_hbm.at[pl.ds(cbase, n_chunks)], sidx_v)
        pltpu.sync_copy(ofs_hbm.at[pl.ds(cbase, n_chunks)], ofs_v)
        iota = lax.iota(jnp.int32, 16)

        def chunk(j, carry):
            pltpu.async_copy(table_hbm.at[sidx_v.at[j]], sup_v, gsem).wait()

            def group(g, carry2):
                r0 = g * 16
                o_vec = ofs_v[j, pl.ds(r0, 16)]
                for l in range(16):
                    r = r0 + l
                    o = o_vec[l]
                    rows = jnp.full((16,), r, jnp.int32)
                    c0 = o + iota
                    v0 = plsc.load_gather(sup_v, [rows, c0])
                    v1 = plsc.load_gather(sup_v, [rows, c0 + 16])
                    out_v[r, pl.ds(0, 16)] = v0
                    out_v[r, pl.ds(16, 16)] = v1
                return carry2

            lax.fori_loop(0, CH // 16, group, 0)
            pltpu.async_copy(
                out_v, out_hbm.at[pl.ds(rbase + j * CH, CH)], ssem
            ).wait()
            return carry

        lax.fori_loop(0, n_chunks, chunk, 0)

    return body


def kernel(token_ids, emb):
    bsz, hist = token_ids.shape
    n_rows = bsz * hist
    d = emb.shape[1]
    per_sup = 128 // d
    n_sup = emb.shape[0] // per_sup
    sup = emb.reshape(n_sup, 128)
    flat = token_ids.astype(jnp.int32).reshape(n_rows)
    sidx = (flat // per_sup).reshape(n_rows // CH, CH)
    ofs = ((flat % per_sup) * d).reshape(n_rows // CH, CH)
    out = _build(n_rows, d, n_sup)(sidx, ofs, sup)
    return out.reshape(bsz, hist, d)


# speedup vs baseline: 1.1873x; 1.1873x over previous
import functools

import jax
import jax.numpy as jnp
from jax import lax
from jax.experimental import pallas as pl
from jax.experimental.pallas import tpu as pltpu
from jax.experimental.pallas import tpu_sc as plsc

NC = 2
NS = 16
NW = NC * NS
CH = 128
NB = 4


@functools.lru_cache(maxsize=None)
def _build(n_rows, d):
    assert n_rows % (NW * CH * NB) == 0
    b_per_w = n_rows // NW
    n_chunks = b_per_w // CH
    n_groups = n_chunks // NB
    mesh = plsc.VectorSubcoreMesh(core_axis_name="c", subcore_axis_name="s")

    @functools.partial(
        pl.kernel,
        mesh=mesh,
        compiler_params=pltpu.CompilerParams(
            needs_layout_passes=False, use_tc_tiling_on_sc=False
        ),
        out_type=jax.ShapeDtypeStruct((n_rows, d), jnp.float32),
        scratch_types=[
            pltpu.VMEM((n_chunks, CH), jnp.int32),
            *[pltpu.VMEM((CH, d), jnp.float32) for _ in range(NB)],
            *[pltpu.SemaphoreType.DMA for _ in range(2 * NB)],
        ],
    )
    def body(idx_hbm, table_hbm, out_hbm, idx_v, *bufs_and_sems):
        bufs = bufs_and_sems[:NB]
        gsems = bufs_and_sems[NB:2 * NB]
        ssems = bufs_and_sems[2 * NB:]
        wid = lax.axis_index("s") * NC + lax.axis_index("c")
        cbase = wid * n_chunks
        rbase = wid * b_per_w
        pltpu.sync_copy(idx_hbm.at[pl.ds(cbase, n_chunks)], idx_v)

        def out_slice(j):
            return out_hbm.at[pl.ds(rbase + j * CH, CH)]

        def group(i, carry):
            for p in range(NB):
                j = i * NB + p

                @pl.when(i > 0)
                def _drain():
                    pltpu.make_async_copy(
                        bufs[p], out_slice(j - NB), ssems[p]
                    ).wait()

                pltpu.async_copy(
                    table_hbm.at[idx_v.at[j]], bufs[p], gsems[p]
                )
            for p in range(NB):
                j = i * NB + p
                pltpu.make_async_copy(
                    table_hbm.at[idx_v.at[j]], bufs[p], gsems[p]
                ).wait()
                pltpu.async_copy(bufs[p], out_slice(j), ssems[p])
            return carry

        lax.fori_loop(0, n_groups, group, 0)
        for p in range(NB):
            j = (n_groups - 1) * NB + p
            pltpu.make_async_copy(bufs[p], out_slice(j), ssems[p]).wait()

    return body


def kernel(token_ids, emb):
    bsz, hist = token_ids.shape
    n_rows = bsz * hist
    d = emb.shape[1]
    idx2d = token_ids.astype(jnp.int32).reshape(n_rows // CH, CH)
    out = _build(n_rows, d)(idx2d, emb)
    return out.reshape(bsz, hist, d)


# speedup vs baseline: 1.8241x; 1.5363x over previous
import functools

import jax
import jax.numpy as jnp
from jax import lax
from jax.experimental import pallas as pl
from jax.experimental.pallas import tpu as pltpu
from jax.experimental.pallas import tpu_sc as plsc

NC = 2
NS = 16
NW = NC * NS
NB = 4


@functools.lru_cache(maxsize=None)
def _build(bsz, hist, d):
    assert bsz % (NW * NB) == 0
    rows_per_w = bsz // NW
    n_groups = rows_per_w // NB
    mesh = plsc.VectorSubcoreMesh(core_axis_name="c", subcore_axis_name="s")

    @functools.partial(
        pl.kernel,
        mesh=mesh,
        compiler_params=pltpu.CompilerParams(
            needs_layout_passes=False, use_tc_tiling_on_sc=False
        ),
        out_type=jax.ShapeDtypeStruct((bsz, hist, d), jnp.float32),
        scratch_types=[
            pltpu.VMEM((rows_per_w, hist), jnp.int32),
            *[pltpu.VMEM((hist, d), jnp.float32) for _ in range(NB)],
            *[pltpu.SemaphoreType.DMA for _ in range(2 * NB)],
        ],
    )
    def body(idx_hbm, table_hbm, out_hbm, idx_v, *bufs_and_sems):
        bufs = bufs_and_sems[:NB]
        gsems = bufs_and_sems[NB:2 * NB]
        ssems = bufs_and_sems[2 * NB:]
        wid = lax.axis_index("s") * NC + lax.axis_index("c")
        b0 = wid * rows_per_w
        pltpu.sync_copy(idx_hbm.at[pl.ds(b0, rows_per_w)], idx_v)

        def group(i, carry):
            for p in range(NB):
                b = i * NB + p

                @pl.when(i > 0)
                def _drain():
                    pltpu.make_async_copy(
                        bufs[p], out_hbm.at[b0 + b - NB], ssems[p]
                    ).wait()

                pltpu.async_copy(
                    table_hbm.at[idx_v.at[b]], bufs[p], gsems[p]
                )
            for p in range(NB):
                b = i * NB + p
                pltpu.make_async_copy(
                    table_hbm.at[idx_v.at[b]], bufs[p], gsems[p]
                ).wait()
                pltpu.async_copy(bufs[p], out_hbm.at[b0 + b], ssems[p])
            return carry

        lax.fori_loop(0, n_groups, group, 0)
        for p in range(NB):
            b = (n_groups - 1) * NB + p
            pltpu.make_async_copy(
                bufs[p], out_hbm.at[b0 + b], ssems[p]
            ).wait()

    return body


def kernel(token_ids, emb):
    bsz, hist = token_ids.shape
    d = emb.shape[1]
    return _build(bsz, hist, d)(token_ids.astype(jnp.int32), emb)
